# baseline (device time: 85062 ns/iter reference)
import jax
import jax.numpy as jnp
from jax import lax
from jax.experimental import pallas as pl
from jax.experimental.pallas import tpu as pltpu

B, S, H, Dh, Dr = 2, 512, 16, 128, 32
D = 2048
DC_SH = 128
DC = 2 * DC_SH
DH_CAT = Dh + Dr
SCALE = (Dh + Dr) ** -0.5

_F32 = jnp.float32
_BF16 = jnp.bfloat16


def _dot(a, b):
    return jnp.dot(a, b, preferred_element_type=_F32)


def _dot_t(a, b):
    return lax.dot_general(a, b, (((1,), (1,)), ((), ())),
                           preferred_element_type=_F32)


_N_WQ_CHUNKS = 8
_WQ_COLS = D // _N_WQ_CHUNKS


def _exchange_and_project(x2d, Wdkv, Wuk, Wuv, Wq, Wqr, Wkr):
    def body(x_ref, wdkv_ref, wuk_ref, wuv_ref, wq_hbm, wqr_ref, wkr_ref,
             qcat_ref, kcat_ref, v_ref,
             x_bf, c_cat, wuk_cat, wuv_cat, wq_stg0, wq_stg1, wqc0, wqc1,
             wq_sem, send_sems, recv_sems):
        wq_stages = (wq_stg0, wq_stg1)
        wqc = (wqc0, wqc1)

        def wq_chunk_copy(j):
            return pltpu.make_async_copy(
                wq_hbm.at[:, pl.ds(j * _WQ_COLS, _WQ_COLS)],
                wq_stages[j % 2], wq_sem)

        wq_chunk_copy(0).start()

        my_x = lax.axis_index("x")
        my_y = lax.axis_index("y")
        my_z = lax.axis_index("z")
        partner = (my_x, 1 - my_y, my_z)

        barrier_sem = pltpu.get_barrier_semaphore()
        pl.semaphore_signal(barrier_sem, inc=1, device_id=partner,
                            device_id_type=pl.DeviceIdType.MESH)
        pl.semaphore_wait(barrier_sem, 1)

        wuk_cat[0:DC_SH, :] = wuk_ref[...].astype(_BF16)
        rdma_wuk = pltpu.make_async_remote_copy(
            src_ref=wuk_cat.at[0:DC_SH, :], dst_ref=wuk_cat.at[DC_SH:DC, :],
            send_sem=send_sems.at[0], recv_sem=recv_sems.at[0],
            device_id=partner, device_id_type=pl.DeviceIdType.MESH)
        rdma_wuk.start()
        wuv_cat[0:DC_SH, :] = wuv_ref[...].astype(_BF16)
        rdma_wuv = pltpu.make_async_remote_copy(
            src_ref=wuv_cat.at[0:DC_SH, :], dst_ref=wuv_cat.at[DC_SH:DC, :],
            send_sem=send_sems.at[1], recv_sem=recv_sems.at[1],
            device_id=partner, device_id_type=pl.DeviceIdType.MESH)
        rdma_wuv.start()

        x_bf[...] = x_ref[...].astype(_BF16)
        c_cat[:, 0:DC_SH] = _dot(x_bf[...],
                                 wdkv_ref[...].astype(_BF16)).astype(_BF16)
        rdma_c = pltpu.make_async_remote_copy(
            src_ref=c_cat.at[:, 0:DC_SH], dst_ref=c_cat.at[:, DC_SH:DC],
            send_sem=send_sems.at[2], recv_sem=recv_sems.at[2],
            device_id=partner, device_id_type=pl.DeviceIdType.MESH)
        rdma_c.start()

        xb = x_bf[...]
        for j in range(_N_WQ_CHUNKS):
            wq_chunk_copy(j).wait()
            if j + 1 < _N_WQ_CHUNKS:
                wq_chunk_copy(j + 1).start()
            wqc[j % 2][...] = wq_stages[j % 2][...].astype(_BF16)
            qchunk = _dot(xb, wqc[j % 2][...]) * SCALE
            for i in range(_WQ_COLS // Dh):
                h = j * (_WQ_COLS // Dh) + i
                qcat_ref[:, h * DH_CAT:h * DH_CAT + Dh] = (
                    qchunk[:, i * Dh:(i + 1) * Dh].astype(_BF16))
        qrfull = (_dot(xb, wqr_ref[...].astype(_BF16)) * SCALE)
        for h in range(H):
            qcat_ref[:, h * DH_CAT + Dh:(h + 1) * DH_CAT] = (
                qrfull[:, h * Dr:(h + 1) * Dr].astype(_BF16))
        krv = _dot(xb, wkr_ref[...].astype(_BF16)).astype(_BF16)

        rdma_wuk.wait()
        rdma_wuv.wait()
        rdma_c.wait()

        kfull = _dot(c_cat[...], wuk_cat[...])
        for h in range(H):
            kcat_ref[:, h * DH_CAT:h * DH_CAT + Dh] = (
                kfull[:, h * Dh:(h + 1) * Dh].astype(_BF16))
            kcat_ref[:, h * DH_CAT + Dh:(h + 1) * DH_CAT] = krv
        v_ref[...] = _dot(c_cat[...], wuv_cat[...]).astype(_BF16)

    return pl.pallas_call(
        body,
        out_shape=[
            jax.ShapeDtypeStruct((B * S, H * DH_CAT), _BF16),
            jax.ShapeDtypeStruct((B * S, H * DH_CAT), _BF16),
            jax.ShapeDtypeStruct((B * S, D), _BF16),
        ],
        in_specs=([pl.BlockSpec(memory_space=pltpu.VMEM)] * 4
                  + [pl.BlockSpec(memory_space=pltpu.MemorySpace.HBM)]
                  + [pl.BlockSpec(memory_space=pltpu.VMEM)] * 2),
        out_specs=[pl.BlockSpec(memory_space=pltpu.VMEM)] * 3,
        scratch_shapes=[
            pltpu.VMEM((B * S, D), _BF16),
            pltpu.VMEM((B * S, DC), _BF16),
            pltpu.VMEM((DC, D), _BF16),
            pltpu.VMEM((DC, D), _BF16),
            pltpu.VMEM((D, _WQ_COLS), _F32),
            pltpu.VMEM((D, _WQ_COLS), _F32),
            pltpu.VMEM((D, _WQ_COLS), _BF16),
            pltpu.VMEM((D, _WQ_COLS), _BF16),
            pltpu.SemaphoreType.DMA,
            pltpu.SemaphoreType.DMA((3,)),
            pltpu.SemaphoreType.DMA((3,)),
        ],
        compiler_params=pltpu.CompilerParams(collective_id=0),
    )(x2d, Wdkv, Wuk, Wuv, Wq, Wqr, Wkr)


_N_WO_CHUNKS = 4
_WO_ROWS = D // _N_WO_CHUNKS


def _attention_out(qcat2d, kcat2d, v2d, Wo):
    def body(q_ref, k_ref, v_ref, wo_hbm, out_ref,
             o_scr, wo_bf, stg0, stg1, dma_sem):
        b = pl.program_id(0)
        stages = (stg0, stg1)

        def wo_chunk_copy(j):
            return pltpu.make_async_copy(
                wo_hbm.at[pl.ds(j * _WO_ROWS, _WO_ROWS), :],
                stages[j % 2], dma_sem)

        @pl.when(b == 0)
        def _():
            wo_chunk_copy(0).start()

        for h in range(H):
            if h % 4 == 2:
                j = h // 4

                @pl.when(b == 0)
                def _(j=j):
                    wo_chunk_copy(j).wait()
                    if j + 1 < _N_WO_CHUNKS:
                        wo_chunk_copy(j + 1).start()
                    wo_bf[j * _WO_ROWS:(j + 1) * _WO_ROWS, :] = (
                        stages[j % 2][...].astype(_BF16))

            q = q_ref[:, h * DH_CAT:(h + 1) * DH_CAT]
            k = k_ref[:, h * DH_CAT:(h + 1) * DH_CAT]
            s = _dot_t(q, k)
            p = jnp.exp(s)
            o_un = _dot(p.astype(_BF16), v_ref[:, h * Dh:(h + 1) * Dh])
            rs = jnp.sum(p, axis=-1, keepdims=True)
            o_scr[:, h * Dh:(h + 1) * Dh] = (o_un * (1.0 / rs)).astype(_BF16)

        out_ref[...] = _dot(o_scr[...], wo_bf[...])

    return pl.pallas_call(
        body,
        grid=(B,),
        in_specs=[
            pl.BlockSpec((S, H * DH_CAT), lambda b: (b, 0)),
            pl.BlockSpec((S, H * DH_CAT), lambda b: (b, 0)),
            pl.BlockSpec((S, H * Dh), lambda b: (b, 0)),
            pl.BlockSpec(memory_space=pltpu.MemorySpace.HBM),
        ],
        out_specs=pl.BlockSpec((S, D), lambda b: (b, 0)),
        out_shape=jax.ShapeDtypeStruct((B * S, D), _F32),
        scratch_shapes=[
            pltpu.VMEM((S, H * Dh), _BF16),
            pltpu.VMEM((D, D), _BF16),
            pltpu.VMEM((_WO_ROWS, D), _F32),
            pltpu.VMEM((_WO_ROWS, D), _F32),
            pltpu.SemaphoreType.DMA,
        ],
    )(qcat2d, kcat2d, v2d, Wo)


def kernel(x, Wdkv, Wuk, Wuv, Wq, Wqr, Wkr, Wo):
    x2d = x.reshape(B * S, D)
    qcat2d, kcat2d, v2d = _exchange_and_project(
        x2d, Wdkv, Wuk, Wuv, Wq, Wqr, Wkr)
    out2d = _attention_out(qcat2d, kcat2d, v2d, Wo)
    return out2d.reshape(B, S, D)


# device time: 67651 ns/iter; 1.2574x vs baseline; 1.2574x over previous
import jax
import jax.numpy as jnp
from jax import lax
from jax.experimental import pallas as pl
from jax.experimental.pallas import tpu as pltpu

B, S, H, Dh, Dr = 2, 512, 16, 128, 32
D = 2048
DC_SH = 128
DC = 2 * DC_SH
DH_CAT = Dh + Dr
SCALE = (Dh + Dr) ** -0.5

_F32 = jnp.float32
_BF16 = jnp.bfloat16


def _dot(a, b):
    return jnp.dot(a, b, preferred_element_type=_F32)


def _dot_t(a, b):
    return lax.dot_general(a, b, (((1,), (1,)), ((), ())),
                           preferred_element_type=_F32)


_N_WQ_CHUNKS = 8
_WQ_COLS = D // _N_WQ_CHUNKS


def _exchange_and_project(x2d, Wdkv, Wuk, Wuv, Wq, Wqr, Wkr):
    def body(x_ref, wdkv_ref, wuk_ref, wuv_ref, wq_ref, wqr_ref, wkr_ref,
             qcat_ref, kcat_ref, v_ref,
             c_cat, wuk_cat, wuv_cat, wqc0, wqc1,
             send_sems, recv_sems):
        wqc = (wqc0, wqc1)
        my_x = lax.axis_index("x")
        my_y = lax.axis_index("y")
        my_z = lax.axis_index("z")
        partner = (my_x, 1 - my_y, my_z)

        barrier_sem = pltpu.get_barrier_semaphore()
        pl.semaphore_signal(barrier_sem, inc=1, device_id=partner,
                            device_id_type=pl.DeviceIdType.MESH)
        pl.semaphore_wait(barrier_sem, 1)

        wuk_cat[0:DC_SH, :] = wuk_ref[...].astype(_BF16)
        rdma_wuk = pltpu.make_async_remote_copy(
            src_ref=wuk_cat.at[0:DC_SH, :], dst_ref=wuk_cat.at[DC_SH:DC, :],
            send_sem=send_sems.at[0], recv_sem=recv_sems.at[0],
            device_id=partner, device_id_type=pl.DeviceIdType.MESH)
        rdma_wuk.start()
        wuv_cat[0:DC_SH, :] = wuv_ref[...].astype(_BF16)
        rdma_wuv = pltpu.make_async_remote_copy(
            src_ref=wuv_cat.at[0:DC_SH, :], dst_ref=wuv_cat.at[DC_SH:DC, :],
            send_sem=send_sems.at[1], recv_sem=recv_sems.at[1],
            device_id=partner, device_id_type=pl.DeviceIdType.MESH)
        rdma_wuv.start()

        c_cat[:, 0:DC_SH] = _dot(x_ref[...],
                                 wdkv_ref[...].astype(_BF16)).astype(_BF16)
        rdma_c = pltpu.make_async_remote_copy(
            src_ref=c_cat.at[:, 0:DC_SH], dst_ref=c_cat.at[:, DC_SH:DC],
            send_sem=send_sems.at[2], recv_sem=recv_sems.at[2],
            device_id=partner, device_id_type=pl.DeviceIdType.MESH)
        rdma_c.start()

        xb = x_ref[...]
        for j in range(_N_WQ_CHUNKS):
            wqc[j % 2][...] = (
                wq_ref[:, j * _WQ_COLS:(j + 1) * _WQ_COLS].astype(_BF16))
            qchunk = _dot(xb, wqc[j % 2][...]) * SCALE
            for i in range(_WQ_COLS // Dh):
                h = j * (_WQ_COLS // Dh) + i
                qcat_ref[:, h * DH_CAT:h * DH_CAT + Dh] = (
                    qchunk[:, i * Dh:(i + 1) * Dh].astype(_BF16))
        qrfull = (_dot(xb, wqr_ref[...].astype(_BF16)) * SCALE)
        for h in range(H):
            qcat_ref[:, h * DH_CAT + Dh:(h + 1) * DH_CAT] = (
                qrfull[:, h * Dr:(h + 1) * Dr].astype(_BF16))
        krv = _dot(xb, wkr_ref[...].astype(_BF16)).astype(_BF16)

        rdma_wuk.wait()
        rdma_wuv.wait()
        rdma_c.wait()

        kfull = _dot(c_cat[...], wuk_cat[...])
        for h in range(H):
            kcat_ref[:, h * DH_CAT:h * DH_CAT + Dh] = (
                kfull[:, h * Dh:(h + 1) * Dh].astype(_BF16))
            kcat_ref[:, h * DH_CAT + Dh:(h + 1) * DH_CAT] = krv
        v_ref[...] = _dot(c_cat[...], wuv_cat[...]).astype(_BF16)

    return pl.pallas_call(
        body,
        out_shape=[
            jax.ShapeDtypeStruct((B * S, H * DH_CAT), _BF16),
            jax.ShapeDtypeStruct((B * S, H * DH_CAT), _BF16),
            jax.ShapeDtypeStruct((B * S, D), _BF16),
        ],
        in_specs=[pl.BlockSpec(memory_space=pltpu.VMEM)] * 7,
        out_specs=[pl.BlockSpec(memory_space=pltpu.VMEM)] * 3,
        scratch_shapes=[
            pltpu.VMEM((B * S, DC), _BF16),
            pltpu.VMEM((DC, D), _BF16),
            pltpu.VMEM((DC, D), _BF16),
            pltpu.VMEM((D, _WQ_COLS), _BF16),
            pltpu.VMEM((D, _WQ_COLS), _BF16),
            pltpu.SemaphoreType.DMA((3,)),
            pltpu.SemaphoreType.DMA((3,)),
        ],
        compiler_params=pltpu.CompilerParams(collective_id=0),
    )(x2d, Wdkv, Wuk, Wuv, Wq, Wqr, Wkr)


_N_WO_CHUNKS = 4
_WO_ROWS = D // _N_WO_CHUNKS


def _attention_out(qcat2d, kcat2d, v2d, Wo):
    def body(q_ref, k_ref, v_ref, wo_hbm, out_ref,
             o_scr, wo_bf, stg0, stg1, dma_sem):
        b = pl.program_id(0)
        stages = (stg0, stg1)

        def wo_chunk_copy(j):
            return pltpu.make_async_copy(
                wo_hbm.at[pl.ds(j * _WO_ROWS, _WO_ROWS), :],
                stages[j % 2], dma_sem)

        @pl.when(b == 0)
        def _():
            wo_chunk_copy(0).start()

        for h in range(H):
            if h % 4 == 2:
                j = h // 4

                @pl.when(b == 0)
                def _(j=j):
                    wo_chunk_copy(j).wait()
                    if j + 1 < _N_WO_CHUNKS:
                        wo_chunk_copy(j + 1).start()
                    wo_bf[j * _WO_ROWS:(j + 1) * _WO_ROWS, :] = (
                        stages[j % 2][...].astype(_BF16))

            q = q_ref[:, h * DH_CAT:(h + 1) * DH_CAT]
            k = k_ref[:, h * DH_CAT:(h + 1) * DH_CAT]
            s = _dot_t(q, k)
            p = jnp.exp(s)
            o_un = _dot(p.astype(_BF16), v_ref[:, h * Dh:(h + 1) * Dh])
            rs = jnp.sum(p, axis=-1, keepdims=True)
            o_scr[:, h * Dh:(h + 1) * Dh] = (o_un * (1.0 / rs)).astype(_BF16)

        out_ref[...] = _dot(o_scr[...], wo_bf[...])

    return pl.pallas_call(
        body,
        grid=(B,),
        in_specs=[
            pl.BlockSpec((S, H * DH_CAT), lambda b: (b, 0)),
            pl.BlockSpec((S, H * DH_CAT), lambda b: (b, 0)),
            pl.BlockSpec((S, H * Dh), lambda b: (b, 0)),
            pl.BlockSpec(memory_space=pltpu.MemorySpace.HBM),
        ],
        out_specs=pl.BlockSpec((S, D), lambda b: (b, 0)),
        out_shape=jax.ShapeDtypeStruct((B * S, D), _F32),
        scratch_shapes=[
            pltpu.VMEM((S, H * Dh), _BF16),
            pltpu.VMEM((D, D), _BF16),
            pltpu.VMEM((_WO_ROWS, D), _F32),
            pltpu.VMEM((_WO_ROWS, D), _F32),
            pltpu.SemaphoreType.DMA,
        ],
    )(qcat2d, kcat2d, v2d, Wo)


def kernel(x, Wdkv, Wuk, Wuv, Wq, Wqr, Wkr, Wo):
    x2d = x.reshape(B * S, D).astype(_BF16)
    qcat2d, kcat2d, v2d = _exchange_and_project(
        x2d, Wdkv, Wuk, Wuv, Wq, Wqr, Wkr)
    out2d = _attention_out(qcat2d, kcat2d, v2d, Wo)
    return out2d.reshape(B, S, D)
